# Initial kernel scaffold; baseline (speedup 1.0000x reference)
import jax
import jax.numpy as jnp
from jax.experimental import pallas as pl


def kernel(x, batch):
    raise NotImplementedError("write your pallas kernel here")


# speedup vs baseline: 1.5843x; 1.5843x over previous
import functools

import jax
import jax.numpy as jnp
from jax import lax
from jax.experimental import pallas as pl
from jax.experimental.pallas import tpu as pltpu
from jax.experimental.pallas import tpu_sc as plsc

N = 320000
D = 128
S = 10000
NW = 32
SPW = 313
OUT_PAD = NW * SPW
C = 256
DV = D // 16
NINF = float("-inf")


def _sc_segmax(xf, batch, starts):
  mesh = plsc.VectorSubcoreMesh(core_axis_name="c", subcore_axis_name="s")

  @functools.partial(
      pl.kernel,
      mesh=mesh,
      out_type=jax.ShapeDtypeStruct((OUT_PAD * D,), jnp.float32),
      scratch_types=[
          pltpu.VMEM((64,), jnp.int32),
          pltpu.VMEM((C * D,), jnp.float32),
          pltpu.VMEM((C + 16,), jnp.int32),
          pltpu.VMEM((SPW * D,), jnp.float32),
      ],
  )
  def k(x_hbm, b_hbm, st_hbm, out_hbm, st_v, xb, bb, ol):
    wid = lax.axis_index("c") * 16 + lax.axis_index("s")
    pltpu.sync_copy(st_hbm, st_v)
    s_lo = wid * SPW

    ninf = jnp.full((16,), NINF, jnp.float32)

    def initb(j, carry):
      ol[pl.ds(j * 16, 16)] = ninf
      return carry

    lax.fori_loop(0, SPW * D // 16, initb, 0)

    sv = st_v[pl.ds(wid, 16)]
    r_lo = sv[0]
    r_hi = sv[1]
    r0 = lax.bitwise_and(r_lo, jnp.int32(-8))
    nch = (r_hi - r0 + (C - 1)) // C

    def chunk(kk, carry):
      start = pl.multiple_of(jnp.minimum(r0 + kk * C, N - C), 8)
      pltpu.sync_copy(x_hbm.at[pl.ds(pl.multiple_of(start * D, 8), C * D)], xb)
      pltpu.sync_copy(b_hbm.at[pl.ds(start, C)], bb.at[pl.ds(0, C)])

      def row(i, c2):
        b = bb[pl.ds(i, 16)][0]
        inb = jnp.logical_and(b >= s_lo, b < s_lo + SPW)

        @pl.when(inb)
        def _():
          off = (b - s_lo) * D
          for j in range(DV):
            ol[pl.ds(off + j * 16, 16)] = jnp.maximum(
                ol[pl.ds(off + j * 16, 16)], xb[pl.ds(i * D + j * 16, 16)])

        return c2

      return lax.fori_loop(0, C, row, carry)

    lax.fori_loop(0, nch, chunk, 0)
    pltpu.sync_copy(ol, out_hbm.at[pl.ds(s_lo * D, SPW * D)])

  return k(xf, batch, starts)


def kernel(x, batch):
  xf = x.reshape(-1)
  bounds = (jnp.arange(NW + 1, dtype=jnp.int32) * SPW)
  starts = jnp.searchsorted(batch, bounds).astype(jnp.int32)
  starts = jnp.concatenate(
      [starts, jnp.full((64 - (NW + 1),), N, jnp.int32)])
  out = _sc_segmax(xf, batch, starts)
  return out.reshape(OUT_PAD, D)[:S]


# speedup vs baseline: 4.5527x; 2.8735x over previous
import functools

import jax
import jax.numpy as jnp
from jax import lax
from jax.experimental import pallas as pl
from jax.experimental.pallas import tpu as pltpu
from jax.experimental.pallas import tpu_sc as plsc

N = 320000
D = 128
S = 10000
NW = 32
SPW = 313
OUT_PAD = NW * SPW
C = 256
G = 16
DV = D // 16
NINF = float("-inf")


def _sc_segmax(xf, batch, starts):
  mesh = plsc.VectorSubcoreMesh(core_axis_name="c", subcore_axis_name="s")

  @functools.partial(
      pl.kernel,
      mesh=mesh,
      out_type=jax.ShapeDtypeStruct((OUT_PAD * D,), jnp.float32),
      scratch_types=[
          pltpu.VMEM((64,), jnp.int32),
          pltpu.VMEM((C * D,), jnp.float32),
          pltpu.VMEM((C * D,), jnp.float32),
          pltpu.VMEM((C,), jnp.int32),
          pltpu.VMEM((C,), jnp.int32),
          pltpu.VMEM((SPW * D,), jnp.float32),
          pltpu.SemaphoreType.DMA,
          pltpu.SemaphoreType.DMA,
          pltpu.SemaphoreType.DMA,
          pltpu.SemaphoreType.DMA,
      ],
  )
  def k(x_hbm, b_hbm, st_hbm, out_hbm,
        st_v, xb0, xb1, bb0, bb1, ol, xs0, xs1, bs0, bs1):
    wid = lax.axis_index("c") * 16 + lax.axis_index("s")
    pltpu.sync_copy(st_hbm, st_v)
    s_lo = wid * SPW

    ninf = jnp.full((16,), NINF, jnp.float32)

    def initb(j, carry):
      ol[pl.ds(j * 16, 16)] = ninf
      return carry

    lax.fori_loop(0, SPW * D // 16, initb, 0)

    sv = st_v[pl.ds(wid, 16)]
    r_lo = sv[0]
    r_hi = sv[1]
    r0 = lax.bitwise_and(r_lo, jnp.int32(-8))
    nch = (r_hi - r0 + (C - 1)) // C
    npair = (nch + 1) // 2

    xbufs = (xb0, xb1)
    bbufs = (bb0, bb1)
    xsems = (xs0, xs1)
    bsems = (bs0, bs1)

    def chunk_slices(kk):
      st = pl.multiple_of(jnp.minimum(r0 + kk * C, N - C), 8)
      return (x_hbm.at[pl.ds(pl.multiple_of(st * D, 8), C * D)],
              b_hbm.at[pl.ds(st, C)])

    def cstart(kk, s):
      xs, bs = chunk_slices(kk)
      pltpu.async_copy(xs, xbufs[s], xsems[s])
      pltpu.async_copy(bs, bbufs[s], bsems[s])

    def cwait(kk, s):
      xs, bs = chunk_slices(kk)
      pltpu.make_async_copy(xs, xbufs[s], xsems[s]).wait()
      pltpu.make_async_copy(bs, bbufs[s], bsems[s]).wait()

    cstart(0, 0)
    cstart(1, 1)

    def flush(cur, runs):
      off = (cur - s_lo) * D
      for j in range(DV):
        ol[pl.ds(off + j * 16, 16)] = jnp.maximum(
            ol[pl.ds(off + j * 16, 16)], runs[j])

    def process(s, carry):
      xb = xbufs[s]
      bb = bbufs[s]

      def group(g, c2):
        cur, runs = c2
        bvec = bb[pl.ds(g * G, G)]
        for l in range(G):
          b = bvec[l]
          inb = jnp.logical_and(b >= s_lo, b < s_lo + SPW)
          isnew = jnp.logical_and(inb, b != cur)

          @pl.when(jnp.logical_and(isnew, cur >= 0))
          def _(cur=cur, runs=runs):
            flush(cur, runs)

          row = [xb[pl.ds(g * (G * D) + l * D + j * 16, 16)]
                 for j in range(DV)]
          runs = tuple(
              jnp.where(inb,
                        jnp.maximum(jnp.where(isnew, ninf, runs[j]), row[j]),
                        runs[j])
              for j in range(DV))
          cur = jnp.where(isnew, b, cur)
        return (cur, runs)

      return lax.fori_loop(0, C // G, group, c2_init := carry)

    def pair(p, carry):
      for s in (0, 1):
        kk = 2 * p + s
        cwait(kk, s)
        carry = process(s, carry)
        cstart(kk + 2, s)
      return carry

    init = (jnp.int32(-1), tuple(ninf for _ in range(DV)))
    cur, runs = lax.fori_loop(0, npair, pair, init)

    cwait(2 * npair, 0)
    cwait(2 * npair + 1, 1)

    @pl.when(cur >= 0)
    def _():
      flush(cur, runs)

    pltpu.sync_copy(ol, out_hbm.at[pl.ds(s_lo * D, SPW * D)])

  return k(xf, batch, starts)


def kernel(x, batch):
  xf = x.reshape(-1)
  bounds = (jnp.arange(NW + 1, dtype=jnp.int32) * SPW)
  starts = jnp.searchsorted(batch, bounds).astype(jnp.int32)
  starts = jnp.concatenate(
      [starts, jnp.full((64 - (NW + 1),), N, jnp.int32)])
  out = _sc_segmax(xf, batch, starts)
  return out.reshape(OUT_PAD, D)[:S]


# speedup vs baseline: 5.7634x; 1.2659x over previous
import functools

import jax
import jax.numpy as jnp
from jax import lax
from jax.experimental import pallas as pl
from jax.experimental.pallas import tpu as pltpu
from jax.experimental.pallas import tpu_sc as plsc

N = 320000
D = 128
S = 10000
NW = 32
SPW = 313
SPW_LAST = S - (NW - 1) * SPW
C = 256
G = 16
DV = D // 16
NINF = float("-inf")


def _sc_segmax(xf, batch, starts):
  mesh = plsc.VectorSubcoreMesh(core_axis_name="c", subcore_axis_name="s")

  @functools.partial(
      pl.kernel,
      mesh=mesh,
      out_type=jax.ShapeDtypeStruct((S * D,), jnp.float32),
      scratch_types=[
          pltpu.VMEM((64,), jnp.int32),
          pltpu.VMEM((C * D,), jnp.float32),
          pltpu.VMEM((C * D,), jnp.float32),
          pltpu.VMEM((C,), jnp.int32),
          pltpu.VMEM((C,), jnp.int32),
          pltpu.VMEM(((SPW + 1) * D,), jnp.float32),
          pltpu.SemaphoreType.DMA,
          pltpu.SemaphoreType.DMA,
          pltpu.SemaphoreType.DMA,
          pltpu.SemaphoreType.DMA,
      ],
  )
  def k(x_hbm, b_hbm, st_hbm, out_hbm,
        st_v, xb0, xb1, bb0, bb1, ol, xs0, xs1, bs0, bs1):
    wid = lax.axis_index("c") * 16 + lax.axis_index("s")
    pltpu.sync_copy(st_hbm, st_v)
    s_lo = wid * SPW

    ninf = jnp.full((16,), NINF, jnp.float32)

    def initb(j, carry):
      ol[pl.ds(j * 16, 16)] = ninf
      return carry

    lax.fori_loop(0, (SPW + 1) * D // 16, initb, 0)

    sv = st_v[pl.ds(wid, 16)]
    r_lo = sv[0]
    r_hi = sv[1]
    r0 = lax.bitwise_and(r_lo, jnp.int32(-8))
    nch = (r_hi - r0 + (C - 1)) // C
    npair = (nch + 1) // 2

    xbufs = (xb0, xb1)
    bbufs = (bb0, bb1)
    xsems = (xs0, xs1)
    bsems = (bs0, bs1)

    def chunk_slices(kk):
      st = pl.multiple_of(jnp.minimum(r0 + kk * C, N - C), 8)
      return (x_hbm.at[pl.ds(pl.multiple_of(st * D, 8), C * D)],
              b_hbm.at[pl.ds(st, C)])

    def cstart(kk, s):
      xs, bs = chunk_slices(kk)
      pltpu.async_copy(xs, xbufs[s], xsems[s])
      pltpu.async_copy(bs, bbufs[s], bsems[s])

    def cwait(kk, s):
      xs, bs = chunk_slices(kk)
      pltpu.make_async_copy(xs, xbufs[s], xsems[s]).wait()
      pltpu.make_async_copy(bs, bbufs[s], bsems[s]).wait()

    cstart(0, 0)
    cstart(1, 1)

    def flush(cur, runs):
      row = jnp.minimum((cur - s_lo).astype(jnp.uint32),
                        jnp.uint32(SPW)).astype(jnp.int32)
      off = row * D
      for j in range(DV):
        ol[pl.ds(off + j * 16, 16)] = jnp.maximum(
            ol[pl.ds(off + j * 16, 16)], runs[j])

    def process(s, carry):
      xb = xbufs[s]
      bb = bbufs[s]

      def group(g, c2):
        cur, runs = c2
        bvec = bb[pl.ds(g * G, G)]
        for l in range(G):
          b = bvec[l]
          isnew = b != cur

          @pl.when(isnew)
          def _(cur=cur, runs=runs):
            flush(cur, runs)

          row = [xb[pl.ds(g * (G * D) + l * D + j * 16, 16)]
                 for j in range(DV)]
          runs = tuple(
              jnp.where(isnew, row[j], jnp.maximum(runs[j], row[j]))
              for j in range(DV))
          cur = jnp.where(isnew, b, cur)
        return (cur, runs)

      return lax.fori_loop(0, C // G, group, carry)

    def pair(p, carry):
      for s in (0, 1):
        kk = 2 * p + s
        cwait(kk, s)
        carry = process(s, carry)
        cstart(kk + 2, s)
      return carry

    init = (jnp.int32(-1), tuple(ninf for _ in range(DV)))
    cur, runs = lax.fori_loop(0, npair, pair, init)

    cwait(2 * npair, 0)
    cwait(2 * npair + 1, 1)

    flush(cur, runs)

    @pl.when(wid < NW - 1)
    def _():
      pltpu.sync_copy(ol.at[pl.ds(0, SPW * D)],
                      out_hbm.at[pl.ds(s_lo * D, SPW * D)])

    @pl.when(wid == NW - 1)
    def _():
      pltpu.sync_copy(ol.at[pl.ds(0, SPW_LAST * D)],
                      out_hbm.at[pl.ds(s_lo * D, SPW_LAST * D)])

  return k(xf, batch, starts)


def kernel(x, batch):
  xf = x.reshape(-1)
  bounds = jnp.arange(NW, dtype=jnp.int32) * SPW
  starts = jnp.sum(batch[:, None] < bounds[None, :], axis=0,
                   dtype=jnp.int32)
  starts = jnp.concatenate(
      [starts, jnp.full((64 - NW,), N, jnp.int32)])
  out = _sc_segmax(xf, batch, starts)
  return out.reshape(S, D)


# speedup vs baseline: 6.3238x; 1.0972x over previous
import functools

import jax
import jax.numpy as jnp
from jax import lax
from jax.experimental import pallas as pl
from jax.experimental.pallas import tpu as pltpu
from jax.experimental.pallas import tpu_sc as plsc

N = 320000
D = 128
S = 10000
NW = 32
SPW = 313
SPW_LAST = S - (NW - 1) * SPW
C = 256
G = 16
DV = D // 16
NINF = float("-inf")


def _sc_segmax(xf, batch, starts):
  mesh = plsc.VectorSubcoreMesh(core_axis_name="c", subcore_axis_name="s")

  @functools.partial(
      pl.kernel,
      mesh=mesh,
      out_type=jax.ShapeDtypeStruct((S * D,), jnp.float32),
      compiler_params=pltpu.CompilerParams(needs_layout_passes=False),
      scratch_types=[
          pltpu.VMEM((64,), jnp.int32),
          pltpu.VMEM((C * D,), jnp.float32),
          pltpu.VMEM((C * D,), jnp.float32),
          pltpu.VMEM((C,), jnp.int32),
          pltpu.VMEM((C,), jnp.int32),
          pltpu.VMEM(((SPW + 1) * D,), jnp.float32),
          pltpu.VMEM((D,), jnp.float32),
          pltpu.SMEM((8,), jnp.int32),
          pltpu.SemaphoreType.DMA,
          pltpu.SemaphoreType.DMA,
          pltpu.SemaphoreType.DMA,
          pltpu.SemaphoreType.DMA,
      ],
  )
  def k(x_hbm, b_hbm, st_hbm, out_hbm,
        st_v, xb0, xb1, bb0, bb1, ol, rbuf, cursm, xs0, xs1, bs0, bs1):
    wid = lax.axis_index("c") * 16 + lax.axis_index("s")
    pltpu.sync_copy(st_hbm, st_v)
    s_lo = wid * SPW

    ninf = jnp.full((16,), NINF, jnp.float32)

    def initb(j, carry):
      ol[pl.ds(j * 16, 16)] = ninf
      return carry

    lax.fori_loop(0, (SPW + 1) * D // 16, initb, 0)

    sv = st_v[pl.ds(wid, 16)]
    r_lo = sv[0]
    r_hi = sv[1]
    r0 = lax.bitwise_and(r_lo, jnp.int32(-8))
    nch = (r_hi - r0 + (C - 1)) // C
    npair = (nch + 1) // 2

    xbufs = (xb0, xb1)
    bbufs = (bb0, bb1)
    xsems = (xs0, xs1)
    bsems = (bs0, bs1)

    def chunk_slices(kk):
      st = pl.multiple_of(jnp.minimum(r0 + kk * C, N - C), 8)
      return (x_hbm.at[pl.ds(pl.multiple_of(st * D, 8), C * D)],
              b_hbm.at[pl.ds(st, C)])

    def cstart(kk, s):
      xs, bs = chunk_slices(kk)
      pltpu.async_copy(xs, xbufs[s], xsems[s])
      pltpu.async_copy(bs, bbufs[s], bsems[s])

    def cwait(kk, s):
      xs, bs = chunk_slices(kk)
      pltpu.make_async_copy(xs, xbufs[s], xsems[s]).wait()
      pltpu.make_async_copy(bs, bbufs[s], bsems[s]).wait()

    cstart(0, 0)
    cstart(1, 1)

    def flush(cur, runs):
      row = jnp.minimum((cur - s_lo).astype(jnp.uint32),
                        jnp.uint32(SPW)).astype(jnp.int32)
      off = row * D
      for j in range(DV):
        ol[pl.ds(off + j * 16, 16)] = jnp.maximum(
            ol[pl.ds(off + j * 16, 16)], runs[j])

    def load_runs():
      return [rbuf[pl.ds(j * 16, 16)] for j in range(DV)]

    def store_runs(runs):
      for j in range(DV):
        rbuf[pl.ds(j * 16, 16)] = runs[j]

    def process(s, carry):
      xb = xbufs[s]
      bb = bbufs[s]

      def group(g, c2):
        bvec = bb[pl.ds(g * G, G)]
        b0 = bvec[0]
        neq = bvec != jnp.full((G,), b0, jnp.int32)
        nneq = plsc.all_reduce_population_count(neq)[0]
        cur = cursm[0]
        base = g * (G * D)

        def tree(j):
          vs = [xb[pl.ds(base + l * D + j * 16, 16)] for l in range(G)]
          while len(vs) > 1:
            vs = [jnp.maximum(vs[i], vs[i + 1])
                  for i in range(0, len(vs), 2)]
          return vs[0]

        @pl.when(jnp.logical_and(nneq == 0, b0 == cur))
        def _():
          runs = load_runs()
          store_runs([jnp.maximum(runs[j], tree(j)) for j in range(DV)])

        @pl.when(jnp.logical_and(nneq == 0, b0 != cur))
        def _():
          flush(cur, load_runs())
          store_runs([tree(j) for j in range(DV)])
          cursm[0] = b0

        @pl.when(nneq != 0)
        def _():
          c3 = cur
          r3 = load_runs()
          for l in range(G):
            b = bvec[l]
            isnew = b != c3

            @pl.when(isnew)
            def _(c3=c3, r3=r3):
              flush(c3, r3)

            row = [xb[pl.ds(base + l * D + j * 16, 16)]
                   for j in range(DV)]
            r3 = [jnp.where(isnew, row[j], jnp.maximum(r3[j], row[j]))
                  for j in range(DV)]
            c3 = jnp.where(isnew, b, c3)
          store_runs(r3)
          cursm[0] = c3

        return c2

      return lax.fori_loop(0, C // G, group, carry)

    def pair(p, carry):
      for s in (0, 1):
        kk = 2 * p + s
        cwait(kk, s)
        carry = process(s, carry)
        cstart(kk + 2, s)
      return carry

    cursm[0] = jnp.int32(-1)
    store_runs([ninf] * DV)
    lax.fori_loop(0, npair, pair, 0)

    cwait(2 * npair, 0)
    cwait(2 * npair + 1, 1)

    flush(cursm[0], load_runs())

    @pl.when(wid < NW - 1)
    def _():
      pltpu.sync_copy(ol.at[pl.ds(0, SPW * D)],
                      out_hbm.at[pl.ds(s_lo * D, SPW * D)])

    @pl.when(wid == NW - 1)
    def _():
      pltpu.sync_copy(ol.at[pl.ds(0, SPW_LAST * D)],
                      out_hbm.at[pl.ds(s_lo * D, SPW_LAST * D)])

  return k(xf, batch, starts)


def kernel(x, batch):
  xf = x.reshape(-1)
  bounds = jnp.arange(NW, dtype=jnp.int32) * SPW
  starts = jnp.sum(batch[:, None] < bounds[None, :], axis=0,
                   dtype=jnp.int32)
  starts = jnp.concatenate(
      [starts, jnp.full((64 - NW,), N, jnp.int32)])
  out = _sc_segmax(xf, batch, starts)
  return out.reshape(S, D)


# speedup vs baseline: 9.1704x; 1.4501x over previous
import functools

import jax
import jax.numpy as jnp
from jax import lax
from jax.experimental import pallas as pl
from jax.experimental.pallas import tpu as pltpu
from jax.experimental.pallas import tpu_sc as plsc

N = 320000
D = 128
S = 10000
NW = 32
SPW = 313
SPW_LAST = S - (NW - 1) * SPW
C = 256
G = 16
DV = D // 16
BOFF = 8
NINF = float("-inf")


def _sc_segmax(xf, batch, starts):
  mesh = plsc.VectorSubcoreMesh(core_axis_name="c", subcore_axis_name="s")

  @functools.partial(
      pl.kernel,
      mesh=mesh,
      out_type=jax.ShapeDtypeStruct((S * D,), jnp.float32),
      compiler_params=pltpu.CompilerParams(needs_layout_passes=False),
      scratch_types=[
          pltpu.VMEM((64,), jnp.int32),
          pltpu.VMEM((C * D,), jnp.float32),
          pltpu.VMEM((C * D,), jnp.float32),
          pltpu.VMEM((BOFF + C + G,), jnp.int32),
          pltpu.VMEM((BOFF + C + G,), jnp.int32),
          pltpu.VMEM((C + G,), jnp.int32),
          pltpu.VMEM(((SPW + 1) * D,), jnp.float32),
          pltpu.SemaphoreType.DMA,
          pltpu.SemaphoreType.DMA,
          pltpu.SemaphoreType.DMA,
          pltpu.SemaphoreType.DMA,
      ],
  )
  def k(x_hbm, b_hbm, st_hbm, out_hbm,
        st_v, xb0, xb1, bb0, bb1, posb, ol, xs0, xs1, bs0, bs1):
    wid = lax.axis_index("c") * 16 + lax.axis_index("s")
    pltpu.sync_copy(st_hbm, st_v)
    s_lo = wid * SPW

    ninf = jnp.full((16,), NINF, jnp.float32)
    iota = jnp.arange(16, dtype=jnp.int32)

    def initb(j, carry):
      ol[pl.ds(j * 16, 16)] = ninf
      return carry

    lax.fori_loop(0, (SPW + 1) * D // 16, initb, 0)

    sv = st_v[pl.ds(wid, 16)]
    r_lo = sv[0]
    r_hi = sv[1]
    r0 = lax.bitwise_and(r_lo, jnp.int32(-8))
    nch = (r_hi - r0 + (C - 1)) // C
    npair = (nch + 1) // 2

    xbufs = (xb0, xb1)
    bbufs = (bb0, bb1)
    xsems = (xs0, xs1)
    bsems = (bs0, bs1)

    def chunk_slices(kk, s):
      st = pl.multiple_of(jnp.minimum(r0 + kk * C, N - C), 8)
      return ((x_hbm.at[pl.ds(pl.multiple_of(st * D, 8), C * D)],
               xbufs[s]),
              (b_hbm.at[pl.ds(st, C)],
               bbufs[s].at[pl.ds(BOFF, C)]))

    def cstart(kk, s):
      (xs_src, xs_dst), (bs_src, bs_dst) = chunk_slices(kk, s)
      pltpu.async_copy(xs_src, xs_dst, xsems[s])
      pltpu.async_copy(bs_src, bs_dst, bsems[s])

    def cwait(kk, s):
      (xs_src, xs_dst), (bs_src, bs_dst) = chunk_slices(kk, s)
      pltpu.make_async_copy(xs_src, xs_dst, xsems[s]).wait()
      pltpu.make_async_copy(bs_src, bs_dst, bsems[s]).wait()

    cstart(0, 0)
    cstart(1, 1)

    def flush(cur, runs):
      row = jnp.minimum((cur - s_lo).astype(jnp.uint32),
                        jnp.uint32(SPW)).astype(jnp.int32)
      off = row * D
      for j in range(DV):
        ol[pl.ds(off + j * 16, 16)] = jnp.maximum(
            ol[pl.ds(off + j * 16, 16)], runs[j])

    def process(s, carry):
      xb = xbufs[s]
      bb = bbufs[s]

      def p1(g, off):
        bvec = bb[pl.ds(BOFF + g * G, G)]
        bprev = bb[pl.ds(BOFF - 1 + g * G, G)]
        neq = bvec != bprev
        idxv = iota + g * G
        plsc.store_compressed(posb.at[pl.ds(off, G)], idxv, mask=neq)
        return off + plsc.all_reduce_population_count(neq)[0]

      nb = lax.fori_loop(0, C // G, p1, jnp.int32(0))

      def interval(t, c3):
        a, cur, runs = c3
        flush(cur, runs)
        curn = bb[pl.ds(BOFF + a, G)][0]
        end = jnp.where(t < nb, posb[pl.ds(t, G)][0], jnp.int32(C))

        def rowb(i, r4):
          return tuple(
              jnp.maximum(r4[j], xb[pl.ds(i * D + j * 16, 16)])
              for j in range(DV))

        runsn = lax.fori_loop(a, end, rowb, tuple(ninf for _ in range(DV)))
        return (end, curn, runsn)

      cur, runs = carry
      _, curn, runsn = lax.fori_loop(
          0, nb + 1, interval, (jnp.int32(0), cur, runs))
      return (curn, runsn)

    def pair(p, carry):
      for s in (0, 1):
        kk = 2 * p + s
        cwait(kk, s)
        carry = process(s, carry)
        cstart(kk + 2, s)
      return carry

    init = (jnp.int32(-1), tuple(ninf for _ in range(DV)))
    cur, runs = lax.fori_loop(0, npair, pair, init)

    cwait(2 * npair, 0)
    cwait(2 * npair + 1, 1)

    flush(cur, runs)

    @pl.when(wid < NW - 1)
    def _():
      pltpu.sync_copy(ol.at[pl.ds(0, SPW * D)],
                      out_hbm.at[pl.ds(s_lo * D, SPW * D)])

    @pl.when(wid == NW - 1)
    def _():
      pltpu.sync_copy(ol.at[pl.ds(0, SPW_LAST * D)],
                      out_hbm.at[pl.ds(s_lo * D, SPW_LAST * D)])

  return k(xf, batch, starts)


def kernel(x, batch):
  xf = x.reshape(-1)
  bounds = jnp.arange(NW, dtype=jnp.int32) * SPW
  starts = jnp.sum(batch[:, None] < bounds[None, :], axis=0,
                   dtype=jnp.int32)
  starts = jnp.concatenate(
      [starts, jnp.full((64 - NW,), N, jnp.int32)])
  out = _sc_segmax(xf, batch, starts)
  return out.reshape(S, D)
